# baseline (speedup 1.0000x reference)
import functools

import jax
import jax.numpy as jnp
from jax import lax
from jax.experimental import pallas as pl
from jax.experimental.pallas import tpu as pltpu
from jax.experimental.pallas import tpu_sc as plsc

_D = 128
_NW = 32
_GRP = 2
_NBUF = 4
_DEPTH = 2


def _make_gather(batch, hist):
    rows_per_w = batch // _NW
    n_chunks = rows_per_w // _GRP
    mesh = plsc.VectorSubcoreMesh(core_axis_name="c", subcore_axis_name="s")

    @functools.partial(
        pl.kernel,
        mesh=mesh,
        compiler_params=pltpu.CompilerParams(use_tc_tiling_on_sc=True),
        out_type=jax.ShapeDtypeStruct((batch, hist, _D), jnp.float32),
        scratch_types=[
            pltpu.VMEM((rows_per_w, hist), jnp.int32),
            pltpu.VMEM((_NBUF, _GRP, hist, _D), jnp.float32),
            pltpu.SemaphoreType.DMA((_NBUF,)),
            pltpu.SemaphoreType.DMA((_NBUF,)),
        ],
    )
    def gather_kernel(table_hbm, idx_hbm, out_hbm, idx_v, rows_v, gsem, psem):
        cid = lax.axis_index("c")
        sid = lax.axis_index("s")
        wid = sid * 2 + cid
        base = wid * rows_per_w
        pltpu.sync_copy(idx_hbm.at[pl.ds(base, rows_per_w)], idx_v)

        def fire_gathers(j, b):
            for r in range(_GRP):
                pltpu.async_copy(
                    table_hbm.at[idx_v.at[j * _GRP + r]],
                    rows_v.at[b].at[r],
                    gsem.at[b],
                )

        def wait_gathers(j, b):
            pltpu.make_async_copy(
                out_hbm.at[pl.ds(base + j * _GRP, _GRP)],
                rows_v.at[b],
                gsem.at[b],
            ).wait()

        def put_desc(j, b):
            return pltpu.make_async_copy(
                rows_v.at[b],
                out_hbm.at[pl.ds(base + j * _GRP, _GRP)],
                psem.at[b],
            )

        for b in range(_DEPTH):
            fire_gathers(b, b)

        def body(j, carry):
            b = lax.rem(j, _NBUF)
            wait_gathers(j, b)
            put_desc(j, b).start()
            jn = j + _DEPTH

            @pl.when(jn < n_chunks)
            def _():
                bn = lax.rem(jn, _NBUF)

                @pl.when(jn >= _NBUF)
                def _():
                    put_desc(jn - _NBUF, bn).wait()

                fire_gathers(jn, bn)

            return carry

        lax.fori_loop(0, n_chunks, body, 0)

        for t in range(_NBUF):
            jo = n_chunks - _NBUF + t
            put_desc(jo, jo % _NBUF).wait()

    return gather_kernel


_NSPLIT = 4


def kernel(input_ids, embed_table):
    batch, hist = input_ids.shape
    idx = input_ids.astype(jnp.int32)
    part = batch // _NSPLIT
    g = _make_gather(part, hist)
    outs = [
        g(embed_table, lax.slice_in_dim(idx, k * part, (k + 1) * part, axis=0))
        for k in range(_NSPLIT)
    ]
    return jnp.concatenate(outs, axis=0)


# speedup vs baseline: 1.5202x; 1.5202x over previous
import functools

import jax
import jax.numpy as jnp
from jax import lax
from jax.experimental import pallas as pl
from jax.experimental.pallas import tpu as pltpu
from jax.experimental.pallas import tpu_sc as plsc

_D = 128
_NW = 32
_GRP = 2
_NBUF = 4
_DEPTH = 2


def _make_gather(batch, hist):
    rows_per_w = batch // _NW
    n_chunks = rows_per_w // _GRP
    mesh = plsc.VectorSubcoreMesh(core_axis_name="c", subcore_axis_name="s")

    hist_pad = (hist + 7) // 8 * 8

    @functools.partial(
        pl.kernel,
        mesh=mesh,
        out_type=jax.ShapeDtypeStruct((batch, hist_pad, _D), jnp.float32),
        scratch_types=[
            pltpu.VMEM((rows_per_w, hist), jnp.int32),
            pltpu.VMEM((_NBUF, _GRP, hist_pad, _D), jnp.float32),
            pltpu.SemaphoreType.DMA((_NBUF,)),
            pltpu.SemaphoreType.DMA((_NBUF,)),
        ],
    )
    def gather_kernel(table_hbm, idx_hbm, out_hbm, idx_v, rows_v, gsem, psem):
        cid = lax.axis_index("c")
        sid = lax.axis_index("s")
        wid = sid * 2 + cid
        base = wid * rows_per_w
        pltpu.sync_copy(idx_hbm.at[pl.ds(base, rows_per_w)], idx_v)

        def gather_desc(j, b, r):
            return pltpu.make_async_copy(
                table_hbm.at[idx_v.at[j * _GRP + r]],
                rows_v.at[b].at[r].at[pl.ds(0, hist)],
                gsem.at[b],
            )

        def fire_gathers(j, b):
            for r in range(_GRP):
                gather_desc(j, b, r).start()

        def wait_gathers(j, b):
            for r in range(_GRP):
                gather_desc(j, b, r).wait()

        def put_desc(j, b):
            return pltpu.make_async_copy(
                rows_v.at[b],
                out_hbm.at[pl.ds(base + j * _GRP, _GRP)],
                psem.at[b],
            )

        for b in range(_DEPTH):
            fire_gathers(b, b)

        def body(j, carry):
            b = lax.rem(j, _NBUF)
            wait_gathers(j, b)
            put_desc(j, b).start()
            jn = j + _DEPTH

            @pl.when(jn < n_chunks)
            def _():
                bn = lax.rem(jn, _NBUF)

                @pl.when(jn >= _NBUF)
                def _():
                    put_desc(jn - _NBUF, bn).wait()

                fire_gathers(jn, bn)

            return carry

        lax.fori_loop(0, n_chunks, body, 0)

        for t in range(_NBUF):
            jo = n_chunks - _NBUF + t
            put_desc(jo, jo % _NBUF).wait()

    return gather_kernel


def kernel(input_ids, embed_table):
    batch, hist = input_ids.shape
    idx = input_ids.astype(jnp.int32)
    out = _make_gather(batch, hist)(embed_table, idx)
    return lax.slice_in_dim(out, 0, hist, axis=1)


# speedup vs baseline: 1.7784x; 1.1699x over previous
import functools

import jax
import jax.numpy as jnp
from jax import lax
from jax.experimental import pallas as pl
from jax.experimental.pallas import tpu as pltpu
from jax.experimental.pallas import tpu_sc as plsc

_D = 128
_NW = 32
_GRP = 2
_NBUF = 4
_DEPTH = 2


def _make_gather(batch, hist):
    rows_per_w = batch // _NW
    n_chunks = rows_per_w // _GRP
    mesh = plsc.VectorSubcoreMesh(core_axis_name="c", subcore_axis_name="s")

    @functools.partial(
        pl.kernel,
        mesh=mesh,
        out_type=jax.ShapeDtypeStruct((batch, hist, _D), jnp.float32),
        scratch_types=[
            pltpu.VMEM((rows_per_w, hist), jnp.int32),
            pltpu.VMEM((_NBUF, _GRP, hist, _D), jnp.float32),
            pltpu.SemaphoreType.DMA((_NBUF,)),
            pltpu.SemaphoreType.DMA((_NBUF,)),
        ],
    )
    def gather_kernel(table_hbm, idx_hbm, out_hbm, idx_v, rows_v, gsem, psem):
        cid = lax.axis_index("c")
        sid = lax.axis_index("s")
        wid = sid * 2 + cid
        base = wid * rows_per_w
        pltpu.sync_copy(idx_hbm.at[pl.ds(base, rows_per_w)], idx_v)

        def fire_gathers(j, b):
            for r in range(_GRP):
                pltpu.async_copy(
                    table_hbm.at[idx_v.at[j * _GRP + r]],
                    rows_v.at[b].at[r],
                    gsem.at[b],
                )

        def wait_gathers(j, b):
            pltpu.make_async_copy(
                out_hbm.at[pl.ds(base + j * _GRP, _GRP)],
                rows_v.at[b],
                gsem.at[b],
            ).wait()

        def put_desc(j, b):
            return pltpu.make_async_copy(
                rows_v.at[b],
                out_hbm.at[pl.ds(base + j * _GRP, _GRP)],
                psem.at[b],
            )

        for b in range(_DEPTH):
            fire_gathers(b, b)

        def body(j, carry):
            b = lax.rem(j, _NBUF)
            wait_gathers(j, b)
            put_desc(j, b).start()
            jn = j + _DEPTH

            @pl.when(jn < n_chunks)
            def _():
                bn = lax.rem(jn, _NBUF)

                @pl.when(jn >= _NBUF)
                def _():
                    put_desc(jn - _NBUF, bn).wait()

                fire_gathers(jn, bn)

            return carry

        lax.fori_loop(0, n_chunks, body, 0)

        for t in range(_NBUF):
            jo = n_chunks - _NBUF + t
            put_desc(jo, jo % _NBUF).wait()

    return gather_kernel


def kernel(input_ids, embed_table):
    batch, hist = input_ids.shape
    idx = input_ids.astype(jnp.int32)
    return _make_gather(batch, hist)(embed_table, idx)


# speedup vs baseline: 3.3455x; 1.8811x over previous
import functools

import jax
import jax.numpy as jnp
from jax import lax
from jax.experimental import pallas as pl
from jax.experimental.pallas import tpu as pltpu
from jax.experimental.pallas import tpu_sc as plsc

_D = 128
_CHUNK = 128
_NW = 32
_NBUF = 6
_DEPTH = 4


def _make_gather(n_rows):
    b_per_w = n_rows // _NW
    n_chunks = b_per_w // _CHUNK
    mesh = plsc.VectorSubcoreMesh(core_axis_name="c", subcore_axis_name="s")

    @functools.partial(
        pl.kernel,
        mesh=mesh,
        out_type=jax.ShapeDtypeStruct((n_rows, _D), jnp.float32),
        scratch_types=[
            pltpu.VMEM((n_chunks, _CHUNK), jnp.int32),
            pltpu.VMEM((_NBUF, _CHUNK, _D), jnp.float32),
            pltpu.SemaphoreType.DMA((_NBUF,)),
            pltpu.SemaphoreType.DMA((_NBUF,)),
        ],
    )
    def gather_kernel(table_hbm, idx_hbm, out_hbm, idx_v, rows_v, gsem, psem):
        cid = lax.axis_index("c")
        sid = lax.axis_index("s")
        wid = sid * 2 + cid
        base = wid * b_per_w
        pltpu.sync_copy(idx_hbm.at[pl.ds(wid * n_chunks, n_chunks)], idx_v)

        def gather_desc(j, b):
            return pltpu.make_async_copy(
                table_hbm.at[idx_v.at[j]], rows_v.at[b], gsem.at[b]
            )

        def put_desc(j, b):
            return pltpu.make_async_copy(
                rows_v.at[b],
                out_hbm.at[pl.ds(base + j * _CHUNK, _CHUNK)],
                psem.at[b],
            )

        for b in range(_DEPTH):
            gather_desc(b, b).start()

        def body(j, carry):
            b = lax.rem(j, _NBUF)
            gather_desc(j, b).wait()
            put_desc(j, b).start()
            jn = j + _DEPTH

            @pl.when(jn < n_chunks)
            def _():
                bn = lax.rem(jn, _NBUF)

                @pl.when(jn >= _NBUF)
                def _():
                    put_desc(jn - _NBUF, bn).wait()

                gather_desc(jn, bn).start()

            return carry

        lax.fori_loop(0, n_chunks, body, 0)

        for t in range(_NBUF):
            jo = n_chunks - _NBUF + t
            put_desc(jo, jo % _NBUF).wait()

    return gather_kernel


def kernel(input_ids, embed_table):
    batch, hist = input_ids.shape
    idx = input_ids.astype(jnp.int32).T.reshape(-1, _CHUNK)
    out = _make_gather(batch * hist)(embed_table, idx)
    return out.reshape(hist, batch, _D).transpose(1, 0, 2)


# speedup vs baseline: 3.3469x; 1.0004x over previous
import functools

import jax
import jax.numpy as jnp
from jax import lax
from jax.experimental import pallas as pl
from jax.experimental.pallas import tpu as pltpu
from jax.experimental.pallas import tpu_sc as plsc

_D = 128
_CHUNK = 128
_NW = 32
_NBUF = 6
_DEPTH = 5


def _make_gather(n_rows):
    b_per_w = n_rows // _NW
    n_chunks = b_per_w // _CHUNK
    mesh = plsc.VectorSubcoreMesh(core_axis_name="c", subcore_axis_name="s")

    @functools.partial(
        pl.kernel,
        mesh=mesh,
        out_type=jax.ShapeDtypeStruct((n_rows, _D), jnp.float32),
        scratch_types=[
            pltpu.VMEM((n_chunks, _CHUNK), jnp.int32),
            pltpu.VMEM((_NBUF, _CHUNK, _D), jnp.float32),
            pltpu.SemaphoreType.DMA((_NBUF,)),
            pltpu.SemaphoreType.DMA((_NBUF,)),
        ],
    )
    def gather_kernel(table_hbm, idx_hbm, out_hbm, idx_v, rows_v, gsem, psem):
        cid = lax.axis_index("c")
        sid = lax.axis_index("s")
        wid = sid * 2 + cid
        base = wid * b_per_w
        pltpu.sync_copy(idx_hbm.at[pl.ds(wid * n_chunks, n_chunks)], idx_v)

        def gather_desc(j, b):
            return pltpu.make_async_copy(
                table_hbm.at[idx_v.at[j]], rows_v.at[b], gsem.at[b]
            )

        def put_desc(j, b):
            return pltpu.make_async_copy(
                rows_v.at[b],
                out_hbm.at[pl.ds(base + j * _CHUNK, _CHUNK)],
                psem.at[b],
            )

        for b in range(_DEPTH):
            gather_desc(b, b).start()

        def body(j, carry):
            b = lax.rem(j, _NBUF)
            gather_desc(j, b).wait()
            put_desc(j, b).start()
            jn = j + _DEPTH

            @pl.when(jn < n_chunks)
            def _():
                bn = lax.rem(jn, _NBUF)

                @pl.when(jn >= _NBUF)
                def _():
                    put_desc(jn - _NBUF, bn).wait()

                gather_desc(jn, bn).start()

            return carry

        lax.fori_loop(0, n_chunks, body, 0)

        for t in range(_NBUF):
            jo = n_chunks - _NBUF + t
            put_desc(jo, jo % _NBUF).wait()

    return gather_kernel


def kernel(input_ids, embed_table):
    batch, hist = input_ids.shape
    idx = input_ids.astype(jnp.int32).T.reshape(-1, _CHUNK)
    out = _make_gather(batch * hist)(embed_table, idx)
    return out.reshape(hist, batch, _D).transpose(1, 0, 2)
